# Initial kernel scaffold; baseline (speedup 1.0000x reference)
import functools

import jax
import jax.numpy as jnp
from jax import lax
from jax.experimental import pallas as pl
from jax.experimental.pallas import tpu as pltpu
from jax.experimental.pallas import tpu_sc as plsc

N = 10000
D = 128
E = 320000
K_HOPS = 4
EPS = 1e-5

NC = 2
NS = 16
NW = NC * NS
NPAD = 10240
RP = NPAD // NS
CH = 256
CROWS = CH // 128
EPW = 10240
NCHUNK = EPW // CH
EPAD = EPW * NW
ERW = EPW // 128


def _tc_pre_body(x_ref, g_ref, bt_ref, w_ref, b_ref, o_ref):
    x = x_ref[...]
    mean = jnp.sum(x, axis=0, keepdims=True) / N
    msq = jnp.sum(x * x, axis=0, keepdims=True) / N
    var = msq - mean * mean
    h = (x - mean) * lax.rsqrt(var + EPS) * g_ref[...] + bt_ref[...]
    h = jnp.tanh(
        lax.dot_general(h, w_ref[...], (((1,), (1,)), ((), ())),
                        preferred_element_type=jnp.float32) + b_ref[...])
    o_ref[...] = h


def _tc_combine_body(p_ref, o_ref):
    o_ref[...] = p_ref[0] + p_ref[1]


def _tc_post_body(p_ref, wsg_ref, bsg_ref, wout_ref, bout_ref, o_ref):
    h = p_ref[0, :N, :] + p_ref[1, :N, :]
    h = jnp.tanh(
        lax.dot_general(h, wsg_ref[...], (((1,), (1,)), ((), ())),
                        preferred_element_type=jnp.float32) + bsg_ref[...])
    o_ref[...] = lax.dot_general(
        h, wout_ref[...], (((1,), (1,)), ((), ())),
        preferred_element_type=jnp.float32) + bout_ref[...]


def _spmm_body(h_hbm, src_hbm, dst_hbm, w_hbm, out_hbm,
               acc, src_v, dst_v, w_v, rows_v, sem):
    cid = lax.axis_index("c")
    sid = lax.axis_index("s")
    wid = sid * NC + cid

    def zrow(i, carry):
        for k in range(D // 16):
            rows_v[i, pl.ds(k * 16, 16)] = jnp.zeros((16,), jnp.float32)
        return carry
    lax.fori_loop(0, CH, zrow, 0)
    pltpu.sync_copy(rows_v, acc.at[pl.ds(sid * RP, CH)])
    pltpu.sync_copy(rows_v, acc.at[pl.ds(sid * RP + CH, CH)])
    pltpu.sync_copy(rows_v.at[pl.ds(0, RP - 2 * CH)],
                    acc.at[pl.ds(sid * RP + 2 * CH, RP - 2 * CH)])
    plsc.subcore_barrier()

    def chunk_body(g, carry):
        er = wid * ERW + g * CROWS
        pltpu.sync_copy(src_hbm.at[pl.ds(er, CROWS)], src_v)
        pltpu.sync_copy(dst_hbm.at[pl.ds(er, CROWS)], dst_v)
        pltpu.sync_copy(w_hbm.at[pl.ds(er, CROWS)], w_v)
        cps = [pltpu.async_copy(h_hbm.at[src_v.at[j]],
                                rows_v.at[pl.ds(j * 128, 128)], sem)
               for j in range(CROWS)]
        for c in cps:
            c.wait()

        def edge(e, carry2):
            j = lax.shift_right_logical(e, 7)
            c = lax.bitwise_and(e, 127)
            wb = plsc.load_gather(
                w_v, [jnp.full((16,), j, jnp.int32),
                      jnp.full((16,), c, jnp.int32)])
            for k in range(D // 16):
                rows_v[e, pl.ds(k * 16, 16)] = rows_v[e, pl.ds(k * 16, 16)] * wb
            return carry2
        lax.fori_loop(0, CH, edge, 0)

        for j in range(CROWS):
            pltpu.sync_copy(rows_v.at[pl.ds(j * 128, 128)],
                            acc.at[dst_v.at[j]], add=True)
        return carry
    lax.fori_loop(0, NCHUNK, chunk_body, 0)

    plsc.subcore_barrier()
    pltpu.sync_copy(acc.at[pl.ds(sid * RP, RP)],
                    out_hbm.at[cid, pl.ds(sid * RP, RP)])


_spmm_kernel = functools.partial(
    pl.kernel,
    out_type=jax.ShapeDtypeStruct((NC, NPAD, D), jnp.float32),
    mesh=plsc.VectorSubcoreMesh(core_axis_name="c", subcore_axis_name="s"),
    scratch_types=[
        pltpu.VMEM_SHARED((NPAD, D), jnp.float32),
        pltpu.VMEM((CROWS, 128), jnp.int32),
        pltpu.VMEM((CROWS, 128), jnp.int32),
        pltpu.VMEM((CROWS, 128), jnp.float32),
        pltpu.VMEM((CH, D), jnp.float32),
        pltpu.SemaphoreType.DMA,
    ],
)(_spmm_body)


def _tc_pre(xpad, g, bt, w, b):
    return pl.pallas_call(
        _tc_pre_body,
        out_shape=jax.ShapeDtypeStruct((NPAD, D), jnp.float32),
    )(xpad, g, bt, w, b)


def _tc_combine(parts):
    return pl.pallas_call(
        _tc_combine_body,
        out_shape=jax.ShapeDtypeStruct((NPAD, D), jnp.float32),
    )(parts)


def _tc_post(parts, wsg, bsg, wout, bout):
    return pl.pallas_call(
        _tc_post_body,
        out_shape=jax.ShapeDtypeStruct((N, D), jnp.float32),
    )(parts, wsg, bsg, wout, bout)


def kernel(x, edge_index, edge_weight, bn_gamma, bn_beta,
           W_in, b_in, W_sg, b_sg, W_out, b_out):
    xpad = jnp.zeros((NPAD, D), jnp.float32).at[:N].set(x)
    pad = EPAD - E
    dst = jnp.concatenate([edge_index[0], jnp.zeros((pad,), jnp.int32)])
    src = jnp.concatenate([edge_index[1], jnp.zeros((pad,), jnp.int32)])
    w = jnp.concatenate([edge_weight, jnp.zeros((pad,), jnp.float32)])
    dst2 = dst.reshape(EPAD // 128, 128)
    src2 = src.reshape(EPAD // 128, 128)
    w2 = w.reshape(EPAD // 128, 128)

    g = bn_gamma.reshape(1, D)
    bt = bn_beta.reshape(1, D)
    b = b_in.reshape(1, D)
    bsg = b_sg.reshape(1, D)
    bout = b_out.reshape(1, D)

    h = _tc_pre(xpad, g, bt, W_in, b)
    parts = None
    for hop in range(K_HOPS):
        parts = _spmm_kernel(h, src2, dst2, w2)
        if hop < K_HOPS - 1:
            h = _tc_combine(parts)
    return _tc_post(parts, W_sg, bsg, W_out, bout)


# speedup vs baseline: 2.0657x; 2.0657x over previous
import functools

import jax
import jax.numpy as jnp
from jax import lax
from jax.experimental import pallas as pl
from jax.experimental.pallas import tpu as pltpu
from jax.experimental.pallas import tpu_sc as plsc

N = 10000
D = 128
E = 320000
K_HOPS = 4
EPS = 1e-5

NC = 2
NS = 16
NW = NC * NS
NPAD = 10240
RP = NPAD // NS
CH = 256
CROWS = CH // 128
EPW = 10240
NCHUNK = EPW // CH
EPAD = EPW * NW
ERW = EPW // 128


def _tc_pre_body(x_ref, g_ref, bt_ref, w_ref, b_ref, o_ref):
    x = x_ref[...]
    mean = jnp.sum(x, axis=0, keepdims=True) / N
    msq = jnp.sum(x * x, axis=0, keepdims=True) / N
    var = msq - mean * mean
    h = (x - mean) * lax.rsqrt(var + EPS) * g_ref[...] + bt_ref[...]
    h = jnp.tanh(
        lax.dot_general(h, w_ref[...], (((1,), (1,)), ((), ())),
                        preferred_element_type=jnp.float32) + b_ref[...])
    o_ref[...] = h


def _tc_combine_body(p_ref, o_ref):
    o_ref[...] = p_ref[0] + p_ref[1]


def _tc_post_body(p_ref, wsg_ref, bsg_ref, wout_ref, bout_ref, o_ref):
    h = p_ref[0, :N, :] + p_ref[1, :N, :]
    h = jnp.tanh(
        lax.dot_general(h, wsg_ref[...], (((1,), (1,)), ((), ())),
                        preferred_element_type=jnp.float32) + bsg_ref[...])
    o_ref[...] = lax.dot_general(
        h, wout_ref[...], (((1,), (1,)), ((), ())),
        preferred_element_type=jnp.float32) + bout_ref[...]


def _spmm_body(h_hbm, src_hbm, dst_hbm, w_hbm, out_hbm,
               acc, src_v, dst_v, w_v, rows_v, sem):
    cid = lax.axis_index("c")
    sid = lax.axis_index("s")
    wid = sid * NC + cid

    def zrow(i, carry):
        for k in range(D // 16):
            rows_v[i, pl.ds(k * 16, 16)] = jnp.zeros((16,), jnp.float32)
        return carry
    lax.fori_loop(0, CH, zrow, 0)
    pltpu.sync_copy(rows_v, acc.at[pl.ds(sid * RP, CH)])
    pltpu.sync_copy(rows_v, acc.at[pl.ds(sid * RP + CH, CH)])
    pltpu.sync_copy(rows_v.at[pl.ds(0, RP - 2 * CH)],
                    acc.at[pl.ds(sid * RP + 2 * CH, RP - 2 * CH)])
    plsc.subcore_barrier()

    def chunk_body(g, carry):
        er = wid * ERW + g * CROWS
        pltpu.sync_copy(src_hbm.at[pl.ds(er, CROWS)], src_v)
        pltpu.sync_copy(dst_hbm.at[pl.ds(er, CROWS)], dst_v)
        pltpu.sync_copy(w_hbm.at[pl.ds(wid * EPW + g * CH, CH)], w_v)
        cps = [pltpu.async_copy(h_hbm.at[src_v.at[j]],
                                rows_v.at[pl.ds(j * 128, 128)], sem)
               for j in range(CROWS)]
        for c in cps:
            c.wait()

        def edge(e, carry2):
            wb = plsc.load_gather(w_v, [jnp.full((16,), e, jnp.int32)])
            for k in range(D // 16):
                rows_v[e, pl.ds(k * 16, 16)] = rows_v[e, pl.ds(k * 16, 16)] * wb
            return carry2
        lax.fori_loop(0, CH, edge, 0)

        for j in range(CROWS):
            pltpu.sync_copy(rows_v.at[pl.ds(j * 128, 128)],
                            acc.at[dst_v.at[j]], add=True)
        return carry
    lax.fori_loop(0, NCHUNK, chunk_body, 0)

    plsc.subcore_barrier()
    pltpu.sync_copy(acc.at[pl.ds(sid * RP, RP)],
                    out_hbm.at[cid, pl.ds(sid * RP, RP)])


_spmm_kernel = functools.partial(
    pl.kernel,
    out_type=jax.ShapeDtypeStruct((NC, NPAD, D), jnp.float32),
    mesh=plsc.VectorSubcoreMesh(core_axis_name="c", subcore_axis_name="s"),
    compiler_params=pltpu.CompilerParams(needs_layout_passes=False),
    scratch_types=[
        pltpu.VMEM_SHARED((NPAD, D), jnp.float32),
        pltpu.VMEM((CROWS, 128), jnp.int32),
        pltpu.VMEM((CROWS, 128), jnp.int32),
        pltpu.VMEM((CH,), jnp.float32),
        pltpu.VMEM((CH, D), jnp.float32),
        pltpu.SemaphoreType.DMA,
    ],
)(_spmm_body)


def _tc_pre(xpad, g, bt, w, b):
    return pl.pallas_call(
        _tc_pre_body,
        out_shape=jax.ShapeDtypeStruct((NPAD, D), jnp.float32),
    )(xpad, g, bt, w, b)


def _tc_combine(parts):
    return pl.pallas_call(
        _tc_combine_body,
        out_shape=jax.ShapeDtypeStruct((NPAD, D), jnp.float32),
    )(parts)


def _tc_post(parts, wsg, bsg, wout, bout):
    return pl.pallas_call(
        _tc_post_body,
        out_shape=jax.ShapeDtypeStruct((N, D), jnp.float32),
    )(parts, wsg, bsg, wout, bout)


def kernel(x, edge_index, edge_weight, bn_gamma, bn_beta,
           W_in, b_in, W_sg, b_sg, W_out, b_out):
    xpad = jnp.zeros((NPAD, D), jnp.float32).at[:N].set(x)
    pad = EPAD - E
    dst = jnp.concatenate([edge_index[0], jnp.zeros((pad,), jnp.int32)])
    src = jnp.concatenate([edge_index[1], jnp.zeros((pad,), jnp.int32)])
    w2 = jnp.concatenate([edge_weight, jnp.zeros((pad,), jnp.float32)])
    dst2 = dst.reshape(EPAD // 128, 128)
    src2 = src.reshape(EPAD // 128, 128)

    g = bn_gamma.reshape(1, D)
    bt = bn_beta.reshape(1, D)
    b = b_in.reshape(1, D)
    bsg = b_sg.reshape(1, D)
    bout = b_out.reshape(1, D)

    h = _tc_pre(xpad, g, bt, W_in, b)
    parts = None
    for hop in range(K_HOPS):
        parts = _spmm_kernel(h, src2, dst2, w2)
        if hop < K_HOPS - 1:
            h = _tc_combine(parts)
    return _tc_post(parts, W_sg, bsg, W_out, bout)


# speedup vs baseline: 5.2273x; 2.5305x over previous
import functools

import jax
import jax.numpy as jnp
from jax import lax
from jax.experimental import pallas as pl
from jax.experimental.pallas import tpu as pltpu
from jax.experimental.pallas import tpu_sc as plsc

N = 10000
D = 128
E = 320000
K_HOPS = 4
EPS = 1e-5

NC = 2
NS = 16
DH = D // NC
NPAD = 10240
RPT = NPAD // NS
CH = 128
EPW = 20480
NCHUNK = EPW // CH
NPAIR = NCHUNK // 2
EPAD = EPW * NS
ERW = EPW // 128


def _tc_pre_body(x_ref, g_ref, bt_ref, w_ref, b_ref, o_ref):
    x = x_ref[...]
    mean = jnp.mean(x, axis=0, keepdims=True)
    var = jnp.mean(x * x, axis=0, keepdims=True) - mean * mean
    h = (x - mean) * lax.rsqrt(var + EPS) * g_ref[...] + bt_ref[...]
    h = jnp.tanh(
        lax.dot_general(h, w_ref[...], (((1,), (1,)), ((), ())),
                        preferred_element_type=jnp.float32) + b_ref[...])
    o_ref[0, :N] = h[:, :DH]
    o_ref[1, :N] = h[:, DH:]
    z = jnp.zeros((NPAD - N, DH), jnp.float32)
    o_ref[0, N:] = z
    o_ref[1, N:] = z


def _tc_post_body(p_ref, wsg_ref, bsg_ref, wout_ref, bout_ref, o_ref):
    h = jnp.concatenate([p_ref[0, :N], p_ref[1, :N]], axis=1)
    h = jnp.tanh(
        lax.dot_general(h, wsg_ref[...], (((1,), (1,)), ((), ())),
                        preferred_element_type=jnp.float32) + bsg_ref[...])
    o_ref[...] = lax.dot_general(
        h, wout_ref[...], (((1,), (1,)), ((), ())),
        preferred_element_type=jnp.float32) + bout_ref[...]


def _spmm_body(h_hbm, src_hbm, dst_hbm, w_hbm, out_hbm,
               acc, src_v, dst_v, wst0, wst1, gbuf0, gbuf1, sbuf0, sbuf1,
               sem_g0, sem_g1, sem_s0, sem_s1, sem_w0, sem_w1):
    cid = lax.axis_index("c")
    sid = lax.axis_index("s")
    gbufs = (gbuf0, gbuf1)
    sbufs = (sbuf0, sbuf1)
    wsts = (wst0, wst1)
    sem_g = (sem_g0, sem_g1)
    sem_s = (sem_s0, sem_s1)
    sem_w = (sem_w0, sem_w1)
    hpl = h_hbm.at[cid]

    pltpu.sync_copy(src_hbm.at[pl.ds(sid * ERW, ERW)], src_v)
    pltpu.sync_copy(dst_hbm.at[pl.ds(sid * ERW, ERW)], dst_v)

    def zrow(i, carry):
        for k in range(DH // 16):
            sbuf0[i, pl.ds(k * 16, 16)] = jnp.zeros((16,), jnp.float32)
        return carry
    lax.fori_loop(0, CH, zrow, 0)
    for r in range(RPT // CH):
        pltpu.sync_copy(sbuf0, acc.at[pl.ds(sid * RPT + r * CH, CH)])
    plsc.subcore_barrier()

    for b in range(2):
        pltpu.async_copy(w_hbm.at[pl.ds(sid * EPW + b * CH, CH)],
                         wsts[b], sem_w[b])
        pltpu.async_copy(hpl.at[src_v.at[b]], gbufs[b], sem_g[b])

    def pair_body(q, carry):
        for b in range(2):
            g = q * 2 + b
            pltpu.make_async_copy(hpl.at[src_v.at[g]],
                                  gbufs[b], sem_g[b]).wait()
            pltpu.make_async_copy(w_hbm.at[pl.ds(sid * EPW + g * CH, CH)],
                                  wsts[b], sem_w[b]).wait()

            @pl.when(q > 0)
            def _():
                pltpu.make_async_copy(sbufs[b], acc.at[dst_v.at[g]],
                                      sem_s[b]).wait()

            def group(u, carry2):
                e0 = u * 16
                w16 = wsts[b][pl.ds(e0, 16)]
                for j in range(16):
                    wb = lax.gather(
                        w16, jnp.full((16, 1), j, jnp.int32),
                        dimension_numbers=lax.GatherDimensionNumbers(
                            offset_dims=(), collapsed_slice_dims=(0,),
                            start_index_map=(0,)),
                        slice_sizes=(1,),
                        mode=lax.GatherScatterMode.PROMISE_IN_BOUNDS)
                    for k in range(DH // 16):
                        sbufs[b][e0 + j, pl.ds(k * 16, 16)] = (
                            gbufs[b][e0 + j, pl.ds(k * 16, 16)] * wb)
                return carry2
            lax.fori_loop(0, CH // 16, group, 0)

            pltpu.async_copy(sbufs[b], acc.at[dst_v.at[g]], sem_s[b],
                             add=True)

            @pl.when(q < NPAIR - 1)
            def _():
                pltpu.async_copy(hpl.at[src_v.at[g + 2]], gbufs[b], sem_g[b])
                pltpu.async_copy(
                    w_hbm.at[pl.ds(sid * EPW + (g + 2) * CH, CH)],
                    wsts[b], sem_w[b])
        return carry
    lax.fori_loop(0, NPAIR, pair_body, 0)
    for b in range(2):
        g = NCHUNK - 2 + b
        pltpu.make_async_copy(sbufs[b], acc.at[dst_v.at[g]],
                              sem_s[b]).wait()

    plsc.subcore_barrier()
    pltpu.sync_copy(acc.at[pl.ds(sid * RPT, RPT)],
                    out_hbm.at[cid, pl.ds(sid * RPT, RPT)])


_spmm_kernel = functools.partial(
    pl.kernel,
    out_type=jax.ShapeDtypeStruct((NC, NPAD, DH), jnp.float32),
    mesh=plsc.VectorSubcoreMesh(core_axis_name="c", subcore_axis_name="s"),
    compiler_params=pltpu.CompilerParams(needs_layout_passes=False, use_tc_tiling_on_sc=False),
    scratch_types=[
        pltpu.VMEM_SHARED((NPAD, DH), jnp.float32),
        pltpu.VMEM((ERW, 128), jnp.int32),
        pltpu.VMEM((ERW, 128), jnp.int32),
        pltpu.VMEM((CH,), jnp.float32),
        pltpu.VMEM((CH,), jnp.float32),
        pltpu.VMEM((CH, DH), jnp.float32),
        pltpu.VMEM((CH, DH), jnp.float32),
        pltpu.VMEM((CH, DH), jnp.float32),
        pltpu.VMEM((CH, DH), jnp.float32),
        pltpu.SemaphoreType.DMA,
        pltpu.SemaphoreType.DMA,
        pltpu.SemaphoreType.DMA,
        pltpu.SemaphoreType.DMA,
        pltpu.SemaphoreType.DMA,
        pltpu.SemaphoreType.DMA,
    ],
)(_spmm_body)


def _tc_pre(x, g, bt, w, b):
    return pl.pallas_call(
        _tc_pre_body,
        out_shape=jax.ShapeDtypeStruct((NC, NPAD, DH), jnp.float32),
    )(x, g, bt, w, b)


def _tc_post(parts, wsg, bsg, wout, bout):
    return pl.pallas_call(
        _tc_post_body,
        out_shape=jax.ShapeDtypeStruct((N, D), jnp.float32),
    )(parts, wsg, bsg, wout, bout)


def kernel(x, edge_index, edge_weight, bn_gamma, bn_beta,
           W_in, b_in, W_sg, b_sg, W_out, b_out):
    pad = EPAD - E
    dst = jnp.concatenate([edge_index[0], jnp.zeros((pad,), jnp.int32)])
    src = jnp.concatenate([edge_index[1], jnp.zeros((pad,), jnp.int32)])
    w2 = jnp.concatenate([edge_weight, jnp.zeros((pad,), jnp.float32)])
    dst2 = dst.reshape(EPAD // 128, 128)
    src2 = src.reshape(EPAD // 128, 128)

    g = bn_gamma.reshape(1, D)
    bt = bn_beta.reshape(1, D)
    b = b_in.reshape(1, D)
    bsg = b_sg.reshape(1, D)
    bout = b_out.reshape(1, D)

    h = _tc_pre(x, g, bt, W_in, b)
    for _ in range(K_HOPS):
        h = _spmm_kernel(h, src2, dst2, w2)
    return _tc_post(h, W_sg, bsg, W_out, bout)
